# Initial kernel scaffold; baseline (speedup 1.0000x reference)
import jax
import jax.numpy as jnp
from jax.experimental import pallas as pl


def kernel(x, y, name_embedding, value_table):
    raise NotImplementedError("write your pallas kernel here")


# speedup vs baseline: 6.5165x; 6.5165x over previous
import functools

import jax
import jax.numpy as jnp
from jax import lax
from jax.experimental import pallas as pl
from jax.experimental.pallas import tpu as pltpu
from jax.experimental.pallas import tpu_sc as plsc

_B = 4096
_DIC = 100
_D = 64
_NE = 6
_DP = _DIC // 2
_NPAIR = _B * _DP
_NC = 2
_NS = 16
_NW = _NC * _NS
_PER_W = _NPAIR // _NW
_CH = 128
_NCH = _PER_W // _CH


def _tab_body(ne_ref, no_ref, vt_ref, out_ref):
    nm_e = ne_ref[...]
    nm_o = no_ref[...]
    for e0 in range(_NE + 1):
        left = nm_e + vt_ref[e0] if e0 < _NE else nm_e
        for e1 in range(_NE + 1):
            right = nm_o + vt_ref[e1] if e1 < _NE else nm_o
            out_ref[e0, e1] = jnp.concatenate([left, right], axis=-1)


def _build_table(name_embedding, value_table):
    out = pl.pallas_call(
        _tab_body,
        out_shape=jax.ShapeDtypeStruct((_NE + 1, _NE + 1, _DP, 2 * _D), jnp.float32),
    )(name_embedding[0::2], name_embedding[1::2], value_table)
    return out.reshape((_NE + 1) * (_NE + 1) * _DP, 2 * _D)


def _sc_body(ctab_h, xe_h, xo_h, dp_h, ys_h, out_h,
             xe_v, xo_v, dp_v, y_v, i_v, stage_v, sem):
    wid = lax.axis_index("s") * _NC + lax.axis_index("c")
    base0 = wid * _PER_W

    def chunk(c, carry):
        base = base0 + c * _CH
        pltpu.sync_copy(xe_h.at[pl.ds(base, _CH)], xe_v.at[0])
        pltpu.sync_copy(xo_h.at[pl.ds(base, _CH)], xo_v.at[0])
        pltpu.sync_copy(dp_h.at[pl.ds(base, _CH)], dp_v.at[0])
        pltpu.sync_copy(ys_h.at[pl.ds(base, _CH)], y_v.at[0])
        for j in range(_CH // 16):
            sl = pl.ds(j * 16, 16)
            xe = xe_v[0, sl]
            xo = xo_v[0, sl]
            dp = dp_v[0, sl]
            yv = y_v[0, sl]
            d0 = dp * 2
            e0 = jnp.where(d0 == yv, _NE, xe)
            e1 = jnp.where(d0 + 1 == yv, _NE, xo)
            i_v[0, sl] = (e0 * (_NE + 1) + e1) * _DP + dp
        pltpu.async_copy(ctab_h.at[i_v.at[0]], stage_v.at[0], sem).wait()
        pltpu.sync_copy(stage_v.at[0], out_h.at[pl.ds(base, _CH)])
        return carry

    lax.fori_loop(0, _NCH, chunk, 0)


def _sc_gather(ctab, xe, xo, dps, ys):
    mesh = plsc.VectorSubcoreMesh(core_axis_name="c", subcore_axis_name="s")
    run = functools.partial(
        pl.kernel,
        out_type=jax.ShapeDtypeStruct((_NPAIR, 2 * _D), jnp.float32),
        mesh=mesh,
        scratch_types=[
            pltpu.VMEM((1, _CH), jnp.int32),
            pltpu.VMEM((1, _CH), jnp.int32),
            pltpu.VMEM((1, _CH), jnp.int32),
            pltpu.VMEM((1, _CH), jnp.int32),
            pltpu.VMEM((1, _CH), jnp.int32),
            pltpu.VMEM((1, _CH, 2 * _D), jnp.float32),
            pltpu.SemaphoreType.DMA,
        ],
    )(_sc_body)
    return run(ctab, xe, xo, dps, ys)


@jax.jit
def kernel(x, y, name_embedding, value_table):
    x = x.astype(jnp.int32)
    y = y.astype(jnp.int32)
    ctab = _build_table(name_embedding, value_table)
    xe = x[:, 0::2].reshape(_NPAIR)
    xo = x[:, 1::2].reshape(_NPAIR)
    dps = jnp.tile(jnp.arange(_DP, dtype=jnp.int32), _B)
    ys = jnp.repeat(y, _DP)
    out = _sc_gather(ctab, xe, xo, dps, ys)
    return out.reshape(_B, _DIC, _D)


# speedup vs baseline: 8.2751x; 1.2699x over previous
import functools

import jax
import jax.numpy as jnp
from jax import lax
from jax.experimental import pallas as pl
from jax.experimental.pallas import tpu as pltpu
from jax.experimental.pallas import tpu_sc as plsc

_B = 4096
_DIC = 100
_D = 64
_NE = 6
_DP = _DIC // 2
_NPAIR = _B * _DP
_NC = 2
_NS = 16
_NW = _NC * _NS
_PER_W = _NPAIR // _NW
_CH = 128
_NCH = _PER_W // _CH


def _tab_body(ne_ref, no_ref, vt_ref, out_ref):
    nm_e = ne_ref[...]
    nm_o = no_ref[...]
    for e0 in range(_NE + 1):
        left = nm_e + vt_ref[e0] if e0 < _NE else nm_e
        for e1 in range(_NE + 1):
            right = nm_o + vt_ref[e1] if e1 < _NE else nm_o
            out_ref[e0, e1] = jnp.concatenate([left, right], axis=-1)


def _build_table(name_embedding, value_table):
    out = pl.pallas_call(
        _tab_body,
        out_shape=jax.ShapeDtypeStruct((_NE + 1, _NE + 1, _DP, 2 * _D), jnp.float32),
    )(name_embedding[0::2], name_embedding[1::2], value_table)
    return out.reshape((_NE + 1) * (_NE + 1) * _DP, 2 * _D)


_NB = 2


def _sc_body(ctab_h, xe_h, xo_h, dp_h, ys_h, out_h,
             xe_v, xo_v, dp_v, y_v, i_v, stage_v, sem_g, sem_s):
    wid = lax.axis_index("s") * _NC + lax.axis_index("c")
    base0 = wid * _PER_W
    pltpu.sync_copy(xe_h.at[pl.ds(base0, _PER_W)], xe_v)
    pltpu.sync_copy(xo_h.at[pl.ds(base0, _PER_W)], xo_v)
    pltpu.sync_copy(dp_h.at[pl.ds(base0, _PER_W)], dp_v)
    pltpu.sync_copy(ys_h.at[pl.ds(base0, _PER_W)], y_v)

    def idx_chunk(c, carry):
        for j in range(_CH // 16):
            sl = pl.ds(c * _CH + j * 16, 16)
            xe = xe_v[sl]
            xo = xo_v[sl]
            dp = dp_v[sl]
            yv = y_v[sl]
            d0 = dp * 2
            e0 = jnp.where(d0 == yv, _NE, xe)
            e1 = jnp.where(d0 + 1 == yv, _NE, xo)
            i_v[c, pl.ds(j * 16, 16)] = (e0 * (_NE + 1) + e1) * _DP + dp
        return carry

    lax.fori_loop(0, _NCH, idx_chunk, 0)

    def start_gather(c, b):
        pltpu.async_copy(ctab_h.at[i_v.at[c]], stage_v.at[b], sem_g)

    def wait_gather(c, b):
        pltpu.make_async_copy(ctab_h.at[i_v.at[c]], stage_v.at[b], sem_g).wait()

    for b in range(_NB):
        start_gather(b, b)

    def outer(t, carry):
        c0 = t * _NB
        for b in range(_NB):
            c = c0 + b
            base = base0 + c * _CH
            wait_gather(c, b)
            pltpu.async_copy(stage_v.at[b], out_h.at[pl.ds(base, _CH)], sem_s)
            pltpu.make_async_copy(
                stage_v.at[b], out_h.at[pl.ds(base, _CH)], sem_s).wait()

            @pl.when(c + _NB < _NCH)
            def _():
                start_gather(c + _NB, b)
        return carry

    lax.fori_loop(0, _NCH // _NB, outer, 0)


def _sc_gather(ctab, xe, xo, dps, ys):
    mesh = plsc.VectorSubcoreMesh(core_axis_name="c", subcore_axis_name="s")
    run = functools.partial(
        pl.kernel,
        out_type=jax.ShapeDtypeStruct((_NPAIR, 2 * _D), jnp.float32),
        mesh=mesh,
        scratch_types=[
            pltpu.VMEM((_PER_W,), jnp.int32),
            pltpu.VMEM((_PER_W,), jnp.int32),
            pltpu.VMEM((_PER_W,), jnp.int32),
            pltpu.VMEM((_PER_W,), jnp.int32),
            pltpu.VMEM((_NCH, _CH), jnp.int32),
            pltpu.VMEM((_NB, _CH, 2 * _D), jnp.float32),
            pltpu.SemaphoreType.DMA,
            pltpu.SemaphoreType.DMA,
        ],
    )(_sc_body)
    return run(ctab, xe, xo, dps, ys)


@jax.jit
def kernel(x, y, name_embedding, value_table):
    x = x.astype(jnp.int32)
    y = y.astype(jnp.int32)
    ctab = _build_table(name_embedding, value_table)
    xe = x[:, 0::2].reshape(_NPAIR)
    xo = x[:, 1::2].reshape(_NPAIR)
    dps = jnp.tile(jnp.arange(_DP, dtype=jnp.int32), _B)
    ys = jnp.repeat(y, _DP)
    out = _sc_gather(ctab, xe, xo, dps, ys)
    return out.reshape(_B, _DIC, _D)
